# Initial kernel scaffold; baseline (speedup 1.0000x reference)
import functools

import jax
import jax.numpy as jnp
from jax import lax
from jax.experimental import pallas as pl
from jax.experimental.pallas import tpu as pltpu
from jax.experimental.pallas import tpu_sc as plsc

N = 100000
E = 1600000
F = 4
PACK = 32
R = N // PACK
NC, NS = 2, 16
NW = NC * NS
EPW = E // NW
CH = 2000
NCHUNK = EPW // CH
NPS = N // NS

_mesh = plsc.VectorSubcoreMesh(core_axis_name="c", subcore_axis_name="s")


def _deg_body(dst_hbm, zeros_hbm, ones_hbm, out_hbm,
              acc_sh, idx_v, ones_v, sem):
    c = lax.axis_index("c")
    s = lax.axis_index("s")
    wid = c * NS + s
    base_n = s * NPS
    pltpu.sync_copy(zeros_hbm.at[pl.ds(base_n, NPS)],
                    acc_sh.at[pl.ds(base_n, NPS)])
    pltpu.sync_copy(ones_hbm, ones_v)
    plsc.subcore_barrier()

    def chunk(ci, carry):
        base = wid * EPW + ci * CH
        pltpu.sync_copy(dst_hbm.at[pl.ds(base, CH)], idx_v)
        pltpu.sync_copy(ones_v, acc_sh.at[idx_v], add=True)
        return carry

    lax.fori_loop(0, NCHUNK, chunk, 0)
    plsc.subcore_barrier()
    pltpu.sync_copy(acc_sh.at[pl.ds(base_n, NPS)],
                    out_hbm.at[c, pl.ds(base_n, NPS)])


def _scat_body(src_hbm, dst_hbm, y_hbm, zeros_hbm, out_hbm,
               y_sh, acc_sh, sidx_v, didx_v, rows_v, sem):
    c = lax.axis_index("c")
    s = lax.axis_index("s")
    wid = c * NS + s
    base_n = s * NPS
    pltpu.sync_copy(zeros_hbm.at[pl.ds(base_n, NPS)],
                    acc_sh.at[pl.ds(base_n, NPS)])
    pltpu.sync_copy(y_hbm.at[pl.ds(base_n, NPS)],
                    y_sh.at[pl.ds(base_n, NPS)])
    plsc.subcore_barrier()

    def chunk(ci, carry):
        base = wid * EPW + ci * CH
        pltpu.sync_copy(src_hbm.at[pl.ds(base, CH)], sidx_v)
        pltpu.sync_copy(dst_hbm.at[pl.ds(base, CH)], didx_v)
        pltpu.async_copy(y_sh.at[sidx_v], rows_v, sem).wait()
        pltpu.sync_copy(rows_v, acc_sh.at[didx_v], add=True)
        return carry

    lax.fori_loop(0, NCHUNK, chunk, 0)
    plsc.subcore_barrier()
    pltpu.sync_copy(acc_sh.at[pl.ds(base_n, NPS)],
                    out_hbm.at[c, pl.ds(base_n, NPS)])


_sc_deg = pl.kernel(
    _deg_body,
    out_type=jax.ShapeDtypeStruct((NC, N, F), jnp.float32),
    mesh=_mesh,
    scratch_types=[
        pltpu.VMEM_SHARED((N, F), jnp.float32),
        pltpu.VMEM((CH,), jnp.int32),
        pltpu.VMEM((CH, F), jnp.float32),
        pltpu.SemaphoreType.DMA,
    ],
)

_sc_scatter = pl.kernel(
    _scat_body,
    out_type=jax.ShapeDtypeStruct((NC, N, F), jnp.float32),
    mesh=_mesh,
    scratch_types=[
        pltpu.VMEM_SHARED((N, F), jnp.float32),
        pltpu.VMEM_SHARED((N, F), jnp.float32),
        pltpu.VMEM((CH,), jnp.int32),
        pltpu.VMEM((CH,), jnp.int32),
        pltpu.VMEM((CH, F), jnp.float32),
        pltpu.SemaphoreType.DMA,
    ],
)


def _dense1_body(xp_ref, k1_ref, da_ref, db_ref, y1_ref, dinv_ref):
    dinv = lax.rsqrt(da_ref[...] + db_ref[...] + 1.0)
    xw = jnp.dot(xp_ref[...], k1_ref[...], preferred_element_type=jnp.float32)
    dinv_ref[...] = dinv
    y1_ref[...] = xw * dinv


def _dense_mid_body(sa_ref, sb_ref, y_ref, dinv_ref, b_ref, k_ref,
                    h_ref, ynext_ref):
    dinv = dinv_ref[...]
    h = jnp.tanh(dinv * (sa_ref[...] + sb_ref[...] + y_ref[...]) + b_ref[...])
    h_ref[...] = h
    ynext_ref[...] = jnp.dot(h, k_ref[...],
                             preferred_element_type=jnp.float32) * dinv


def _dense_out_body(sa_ref, sb_ref, y_ref, dinv_ref, b_ref, kc_ref, bc_ref,
                    h_ref, out_ref):
    dinv = dinv_ref[...]
    h = jnp.tanh(dinv * (sa_ref[...] + sb_ref[...] + y_ref[...]) + b_ref[...])
    h_ref[...] = h
    out_ref[...] = jnp.dot(h, kc_ref[...],
                           preferred_element_type=jnp.float32) + bc_ref[...]


_p128 = jax.ShapeDtypeStruct((R, 128), jnp.float32)

_dense1 = pl.pallas_call(_dense1_body, out_shape=(_p128, _p128))
_dense_mid = pl.pallas_call(_dense_mid_body, out_shape=(_p128, _p128))
_dense_out = pl.pallas_call(_dense_out_body, out_shape=(_p128, _p128))


def kernel(x, edge_index, W1, b1, W2, b2, W3, b3, Wc, bc):
    ei = edge_index.astype(jnp.int32)
    src, dst = ei[0], ei[1]

    eye = jnp.eye(PACK, dtype=jnp.float32)
    K1 = jnp.kron(eye, W1)
    K2 = jnp.kron(eye, W2)
    K3 = jnp.kron(eye, jnp.pad(W3, ((0, 0), (0, 2))))
    Kc = jnp.kron(eye, jnp.pad(Wc, ((0, 2), (0, 0))))
    b1p = jnp.tile(b1, PACK)[None]
    b2p = jnp.tile(b2, PACK)[None]
    b3p = jnp.tile(jnp.pad(b3, (0, 2)), PACK)[None]
    bcp = jnp.tile(bc, PACK)[None]
    xp = x.reshape(R, PACK * x.shape[1])
    zeros = jnp.zeros((N, F), jnp.float32)
    ones = jnp.ones((CH, F), jnp.float32)

    deg = _sc_deg(dst, zeros, ones)
    da, db = deg[0].reshape(R, 128), deg[1].reshape(R, 128)

    y1, dinv = _dense1(xp, K1, da, db)
    s1 = _sc_scatter(src, dst, y1.reshape(N, F), zeros)
    h1, y2 = _dense_mid(s1[0].reshape(R, 128), s1[1].reshape(R, 128),
                        y1, dinv, b1p, K2)
    s2 = _sc_scatter(src, dst, y2.reshape(N, F), zeros)
    h2, y3 = _dense_mid(s2[0].reshape(R, 128), s2[1].reshape(R, 128),
                        y2, dinv, b2p, K3)
    s3 = _sc_scatter(src, dst, y3.reshape(N, F), zeros)
    h3, outp = _dense_out(s3[0].reshape(R, 128), s3[1].reshape(R, 128),
                          y3, dinv, b3p, Kc, bcp)

    return (outp.reshape(N, F),
            h1.reshape(N, F),
            h2.reshape(N, F),
            h3.reshape(N, F)[:, :2])


# speedup vs baseline: 25.4516x; 25.4516x over previous
import functools

import jax
import jax.numpy as jnp
from jax import lax
from jax.experimental import pallas as pl
from jax.experimental.pallas import tpu as pltpu
from jax.experimental.pallas import tpu_sc as plsc

N = 100000
E = 1600000
F = 4
PACK = 32
R = N // PACK
NC, NS = 2, 16
NW = NC * NS
EPW = E // NW
CH = 2000
NCHUNK = EPW // CH
NPS_A = 6256
NPS_B = N - (NS - 1) * NPS_A

_mesh = plsc.VectorSubcoreMesh(core_axis_name="c", subcore_axis_name="s")


def _striped_copy(s, src, dst):
    base = pl.multiple_of(s * NPS_A, 8)

    @pl.when(s < NS - 1)
    def _():
        pltpu.sync_copy(src.at[pl.ds(base, NPS_A)], dst.at[pl.ds(base, NPS_A)])

    @pl.when(s == NS - 1)
    def _():
        b = (NS - 1) * NPS_A
        pltpu.sync_copy(src.at[pl.ds(b, NPS_B)], dst.at[pl.ds(b, NPS_B)])


def _deg_body(dst_hbm, zeros_hbm, ones_hbm, out_hbm,
              acc_sh, idx_v, ones_v, sem):
    c = lax.axis_index("c")
    s = lax.axis_index("s")
    wid = c * NS + s
    _striped_copy(s, zeros_hbm, acc_sh)
    pltpu.sync_copy(ones_hbm, ones_v)
    plsc.subcore_barrier()

    def chunk(ci, carry):
        base = wid * EPW + ci * CH
        pltpu.sync_copy(dst_hbm.at[pl.ds(base, CH)], idx_v)
        pltpu.sync_copy(ones_v, acc_sh.at[idx_v], add=True)
        return carry

    lax.fori_loop(0, NCHUNK, chunk, 0)
    plsc.subcore_barrier()
    _striped_copy(s, acc_sh, out_hbm.at[c])


def _scat_body(src_hbm, dst_hbm, y_hbm, zeros_hbm, out_hbm,
               y_sh, acc_sh, sidx_v, didx_v, rows_v, sem):
    c = lax.axis_index("c")
    s = lax.axis_index("s")
    wid = c * NS + s
    _striped_copy(s, zeros_hbm, acc_sh)
    _striped_copy(s, y_hbm, y_sh)
    plsc.subcore_barrier()

    def chunk(ci, carry):
        base = wid * EPW + ci * CH
        pltpu.sync_copy(src_hbm.at[pl.ds(base, CH)], sidx_v)
        pltpu.sync_copy(dst_hbm.at[pl.ds(base, CH)], didx_v)
        pltpu.async_copy(y_sh.at[sidx_v], rows_v, sem).wait()
        pltpu.sync_copy(rows_v, acc_sh.at[didx_v], add=True)
        return carry

    lax.fori_loop(0, NCHUNK, chunk, 0)
    plsc.subcore_barrier()
    _striped_copy(s, acc_sh, out_hbm.at[c])


_sc_params = pltpu.CompilerParams(use_tc_tiling_on_sc=False)

_sc_deg = pl.kernel(
    _deg_body,
    out_type=jax.ShapeDtypeStruct((NC, N, F), jnp.float32),
    mesh=_mesh,
    compiler_params=_sc_params,
    scratch_types=[
        pltpu.VMEM_SHARED((N, F), jnp.float32),
        pltpu.VMEM((CH,), jnp.int32),
        pltpu.VMEM((CH, F), jnp.float32),
        pltpu.SemaphoreType.DMA,
    ],
)

_sc_scatter = pl.kernel(
    _scat_body,
    out_type=jax.ShapeDtypeStruct((NC, N, F), jnp.float32),
    mesh=_mesh,
    compiler_params=_sc_params,
    scratch_types=[
        pltpu.VMEM_SHARED((N, F), jnp.float32),
        pltpu.VMEM_SHARED((N, F), jnp.float32),
        pltpu.VMEM((CH,), jnp.int32),
        pltpu.VMEM((CH,), jnp.int32),
        pltpu.VMEM((CH, F), jnp.float32),
        pltpu.SemaphoreType.DMA,
    ],
)


def _dense1_body(xp_ref, k1_ref, da_ref, db_ref, y1_ref, dinv_ref):
    dinv = lax.rsqrt(da_ref[...] + db_ref[...] + 1.0)
    xw = jnp.dot(xp_ref[...], k1_ref[...], preferred_element_type=jnp.float32)
    dinv_ref[...] = dinv
    y1_ref[...] = xw * dinv


def _dense_mid_body(sa_ref, sb_ref, y_ref, dinv_ref, b_ref, k_ref,
                    h_ref, ynext_ref):
    dinv = dinv_ref[...]
    h = jnp.tanh(dinv * (sa_ref[...] + sb_ref[...] + y_ref[...]) + b_ref[...])
    h_ref[...] = h
    ynext_ref[...] = jnp.dot(h, k_ref[...],
                             preferred_element_type=jnp.float32) * dinv


def _dense_out_body(sa_ref, sb_ref, y_ref, dinv_ref, b_ref, kc_ref, bc_ref,
                    h_ref, out_ref):
    dinv = dinv_ref[...]
    h = jnp.tanh(dinv * (sa_ref[...] + sb_ref[...] + y_ref[...]) + b_ref[...])
    h_ref[...] = h
    out_ref[...] = jnp.dot(h, kc_ref[...],
                           preferred_element_type=jnp.float32) + bc_ref[...]


_p128 = jax.ShapeDtypeStruct((R, 128), jnp.float32)

_dense1 = pl.pallas_call(_dense1_body, out_shape=(_p128, _p128))
_dense_mid = pl.pallas_call(_dense_mid_body, out_shape=(_p128, _p128))
_dense_out = pl.pallas_call(_dense_out_body, out_shape=(_p128, _p128))


def kernel(x, edge_index, W1, b1, W2, b2, W3, b3, Wc, bc):
    ei = edge_index.astype(jnp.int32)
    src, dst = ei[0], ei[1]

    eye = jnp.eye(PACK, dtype=jnp.float32)
    K1 = jnp.kron(eye, W1)
    K2 = jnp.kron(eye, W2)
    K3 = jnp.kron(eye, jnp.pad(W3, ((0, 0), (0, 2))))
    Kc = jnp.kron(eye, jnp.pad(Wc, ((0, 2), (0, 0))))
    b1p = jnp.tile(b1, PACK)[None]
    b2p = jnp.tile(b2, PACK)[None]
    b3p = jnp.tile(jnp.pad(b3, (0, 2)), PACK)[None]
    bcp = jnp.tile(bc, PACK)[None]
    xp = x.reshape(R, PACK * x.shape[1])
    zeros = jnp.zeros((N, F), jnp.float32)
    ones = jnp.ones((CH, F), jnp.float32)

    deg = _sc_deg(dst, zeros, ones)
    da, db = deg[0].reshape(R, 128), deg[1].reshape(R, 128)

    y1, dinv = _dense1(xp, K1, da, db)
    s1 = _sc_scatter(src, dst, y1.reshape(N, F), zeros)
    h1, y2 = _dense_mid(s1[0].reshape(R, 128), s1[1].reshape(R, 128),
                        y1, dinv, b1p, K2)
    s2 = _sc_scatter(src, dst, y2.reshape(N, F), zeros)
    h2, y3 = _dense_mid(s2[0].reshape(R, 128), s2[1].reshape(R, 128),
                        y2, dinv, b2p, K3)
    s3 = _sc_scatter(src, dst, y3.reshape(N, F), zeros)
    h3, outp = _dense_out(s3[0].reshape(R, 128), s3[1].reshape(R, 128),
                          y3, dinv, b3p, Kc, bcp)

    return (outp.reshape(N, F),
            h1.reshape(N, F),
            h2.reshape(N, F),
            h3.reshape(N, F)[:, :2])


# speedup vs baseline: 28.8258x; 1.1326x over previous
import functools

import jax
import jax.numpy as jnp
from jax import lax
from jax.experimental import pallas as pl
from jax.experimental.pallas import tpu as pltpu
from jax.experimental.pallas import tpu_sc as plsc

N = 100000
E = 1600000
F = 4
PACK = 32
NP = 102400
RP = NP // PACK
RX = N // PACK
NC, NS = 2, 16
NW = NC * NS
EPW = E // NW
CH = 2000
NCHUNK = EPW // CH
NPS = NP // NS

_mesh = plsc.VectorSubcoreMesh(core_axis_name="c", subcore_axis_name="s")


def _deg_body(ei_hbm, zeros_hbm, ones_hbm, out_hbm,
              acc_sh, idx_v, ones_v, sem):
    c = lax.axis_index("c")
    s = lax.axis_index("s")
    wid = c * NS + s
    b = pl.multiple_of(s * NPS, 8)
    pltpu.sync_copy(zeros_hbm.at[pl.ds(b, NPS)], acc_sh.at[pl.ds(b, NPS)])
    pltpu.sync_copy(ones_hbm, ones_v)
    plsc.subcore_barrier()

    def chunk(ci, carry):
        base = wid * EPW + ci * CH
        pltpu.sync_copy(ei_hbm.at[pl.ds(E + base, CH)], idx_v)
        pltpu.sync_copy(ones_v, acc_sh.at[idx_v], add=True)
        return carry

    lax.fori_loop(0, NCHUNK, chunk, 0)
    plsc.subcore_barrier()
    pltpu.sync_copy(acc_sh.at[pl.ds(b, NPS)], out_hbm.at[c, pl.ds(b, NPS)])


def _scat_body(ei_hbm, y_hbm, zeros_hbm, out_hbm,
               y_sh, acc_sh, sidx_v, didx_v, rows_v, sem):
    c = lax.axis_index("c")
    s = lax.axis_index("s")
    wid = c * NS + s
    b = pl.multiple_of(s * NPS, 8)
    pltpu.sync_copy(zeros_hbm.at[pl.ds(b, NPS)], acc_sh.at[pl.ds(b, NPS)])
    pltpu.sync_copy(y_hbm.at[pl.ds(b, NPS)], y_sh.at[pl.ds(b, NPS)])
    plsc.subcore_barrier()

    def chunk(ci, carry):
        base = wid * EPW + ci * CH
        pltpu.sync_copy(ei_hbm.at[pl.ds(base, CH)], sidx_v)
        pltpu.sync_copy(ei_hbm.at[pl.ds(E + base, CH)], didx_v)
        pltpu.async_copy(y_sh.at[sidx_v], rows_v, sem).wait()
        pltpu.sync_copy(rows_v, acc_sh.at[didx_v], add=True)
        return carry

    lax.fori_loop(0, NCHUNK, chunk, 0)
    plsc.subcore_barrier()
    pltpu.sync_copy(acc_sh.at[pl.ds(b, NPS)], out_hbm.at[c, pl.ds(b, NPS)])


_sc_params = pltpu.CompilerParams(use_tc_tiling_on_sc=False)

_sc_deg = pl.kernel(
    _deg_body,
    out_type=jax.ShapeDtypeStruct((NC, NP, F), jnp.float32),
    mesh=_mesh,
    compiler_params=_sc_params,
    scratch_types=[
        pltpu.VMEM_SHARED((NP, F), jnp.float32),
        pltpu.VMEM((CH,), jnp.int32),
        pltpu.VMEM((CH, F), jnp.float32),
        pltpu.SemaphoreType.DMA,
    ],
)

_sc_scatter = pl.kernel(
    _scat_body,
    out_type=jax.ShapeDtypeStruct((NC, NP, F), jnp.float32),
    mesh=_mesh,
    compiler_params=_sc_params,
    scratch_types=[
        pltpu.VMEM_SHARED((NP, F), jnp.float32),
        pltpu.VMEM_SHARED((NP, F), jnp.float32),
        pltpu.VMEM((CH,), jnp.int32),
        pltpu.VMEM((CH,), jnp.int32),
        pltpu.VMEM((CH, F), jnp.float32),
        pltpu.SemaphoreType.DMA,
    ],
)


def _dense1_body(xp_ref, k1_ref, deg_ref, y1_ref, dinv_ref):
    dinv = lax.rsqrt(deg_ref[0] + deg_ref[1] + 1.0)
    dinv_ref[...] = dinv
    xw = jnp.dot(xp_ref[...], k1_ref[...], preferred_element_type=jnp.float32)
    y1_ref[...] = jnp.concatenate(
        [xw, jnp.zeros((RP - RX, 128), jnp.float32)], axis=0) * dinv


def _dense_mid_body(s_ref, y_ref, dinv_ref, b_ref, k_ref, h_ref, ynext_ref):
    dinv = dinv_ref[...]
    h = jnp.tanh(dinv * (s_ref[0] + s_ref[1] + y_ref[...]) + b_ref[...])
    h_ref[...] = h
    ynext_ref[...] = jnp.dot(h, k_ref[...],
                             preferred_element_type=jnp.float32) * dinv


def _dense_out_body(s_ref, y_ref, dinv_ref, b_ref, kc_ref, bc_ref,
                    h_ref, out_ref):
    dinv = dinv_ref[...]
    h = jnp.tanh(dinv * (s_ref[0] + s_ref[1] + y_ref[...]) + b_ref[...])
    h_ref[...] = h
    out_ref[...] = jnp.dot(h, kc_ref[...],
                           preferred_element_type=jnp.float32) + bc_ref[...]


_p128 = jax.ShapeDtypeStruct((RP, 128), jnp.float32)

_dense1 = pl.pallas_call(_dense1_body, out_shape=(_p128, _p128))
_dense_mid = pl.pallas_call(_dense_mid_body, out_shape=(_p128, _p128))
_dense_out = pl.pallas_call(_dense_out_body, out_shape=(_p128, _p128))


def kernel(x, edge_index, W1, b1, W2, b2, W3, b3, Wc, bc):
    eif = edge_index.astype(jnp.int32).reshape(2 * E)

    eye = jnp.eye(PACK, dtype=jnp.float32)
    K1 = jnp.kron(eye, W1)
    K2 = jnp.kron(eye, W2)
    K3 = jnp.kron(eye, jnp.pad(W3, ((0, 0), (0, 2))))
    Kc = jnp.kron(eye, jnp.pad(Wc, ((0, 2), (0, 0))))
    b1p = jnp.tile(b1, PACK)[None]
    b2p = jnp.tile(b2, PACK)[None]
    b3p = jnp.tile(jnp.pad(b3, (0, 2)), PACK)[None]
    bcp = jnp.tile(bc, PACK)[None]
    xp = x.reshape(RX, PACK * x.shape[1])
    zeros = jnp.zeros((NP, F), jnp.float32)
    ones = jnp.ones((CH, F), jnp.float32)

    deg = _sc_deg(eif, zeros, ones).reshape(NC, RP, 128)
    y1, dinv = _dense1(xp, K1, deg)
    s1 = _sc_scatter(eif, y1.reshape(NP, F), zeros).reshape(NC, RP, 128)
    h1, y2 = _dense_mid(s1, y1, dinv, b1p, K2)
    s2 = _sc_scatter(eif, y2.reshape(NP, F), zeros).reshape(NC, RP, 128)
    h2, y3 = _dense_mid(s2, y2, dinv, b2p, K3)
    s3 = _sc_scatter(eif, y3.reshape(NP, F), zeros).reshape(NC, RP, 128)
    h3, outp = _dense_out(s3, y3, dinv, b3p, Kc, bcp)

    return (outp.reshape(NP, F)[:N],
            h1.reshape(NP, F)[:N],
            h2.reshape(NP, F)[:N],
            h3.reshape(NP, F)[:N, :2])


# speedup vs baseline: 37.1193x; 1.2877x over previous
import functools

import jax
import jax.numpy as jnp
from jax import lax
from jax.experimental import pallas as pl
from jax.experimental.pallas import tpu as pltpu
from jax.experimental.pallas import tpu_sc as plsc

N = 100000
E = 1600000
F = 4
PACK = 32
NP = 102400
RP = NP // PACK
RX = N // PACK
DIN = 34
NC, NS = 2, 16
NW = NC * NS
EPW = E // NW
CH = 2000
NCHUNK = EPW // CH
NPS = NP // NS
RPS = RP // NS
RCH = 8
NCHR = RPS // RCH
CROWS = RCH * PACK

_mesh = plsc.VectorSubcoreMesh(core_axis_name="c", subcore_axis_name="s")


def _iota16():
    i = lax.iota(jnp.int32, 16)
    return i >> 2, i & 3


def _unpack_chunk(pk_c, rows_c):
    r0, c0 = _iota16()
    for r in range(RCH):
        for j in range(8):
            v = pk_c[r, pl.ds(j * 16, 16)]
            plsc.store_scatter(rows_c, [32 * r + 4 * j + r0, c0], v)


def _pack_chunk(rows_c, pk_c):
    r0, c0 = _iota16()
    for r in range(RCH):
        for j in range(8):
            v = plsc.load_gather(rows_c, [32 * r + 4 * j + r0, c0])
            pk_c[r, pl.ds(j * 16, 16)] = v


def _stage_packed_to_rows(src_hbm_rowslice_fn, rows_dst, pk_c, rows_c):
    def stage(q, carry):
        pltpu.sync_copy(src_hbm_rowslice_fn(q), pk_c)
        _unpack_chunk(pk_c, rows_c)
        pltpu.sync_copy(rows_c, rows_dst(q))
        return carry

    lax.fori_loop(0, NCHR, stage, 0)


def _write_rows_to_packed(rows_src, dst_hbm_rowslice_fn, pk_c, rows_c):
    def wb(q, carry):
        pltpu.sync_copy(rows_src(q), rows_c)
        _pack_chunk(rows_c, pk_c)
        pltpu.sync_copy(pk_c, dst_hbm_rowslice_fn(q))
        return carry

    lax.fori_loop(0, NCHR, wb, 0)


def _zero_acc(b, zrows_v, acc_sh):
    for k in range(NPS // CH):
        pltpu.sync_copy(zrows_v, acc_sh.at[pl.ds(b + k * CH, CH)])
    rem = NPS - (NPS // CH) * CH
    if rem:
        pltpu.sync_copy(zrows_v.at[pl.ds(0, rem)],
                        acc_sh.at[pl.ds(b + NPS - rem, rem)])


def _scat_body(ei_hbm, y_hbm, zrows_hbm, out_hbm,
               y_sh, acc_sh, sidx_v, didx_v, rows_v, pk_c, rows_c, sem):
    c = lax.axis_index("c")
    s = lax.axis_index("s")
    wid = c * NS + s
    b = pl.multiple_of(s * NPS, 8)
    rb = pl.multiple_of(s * RPS, 8)
    pltpu.sync_copy(zrows_hbm, rows_v)
    _zero_acc(b, rows_v, acc_sh)
    _stage_packed_to_rows(
        lambda q: y_hbm.at[pl.ds(rb + q * RCH, RCH)],
        lambda q: y_sh.at[pl.ds(b + q * CROWS, CROWS)],
        pk_c, rows_c)
    plsc.subcore_barrier()

    def chunk(ci, carry):
        base = wid * EPW + ci * CH
        pltpu.sync_copy(ei_hbm.at[pl.ds(base, CH)], sidx_v)
        pltpu.sync_copy(ei_hbm.at[pl.ds(E + base, CH)], didx_v)
        pltpu.async_copy(y_sh.at[sidx_v], rows_v, sem).wait()
        pltpu.sync_copy(rows_v, acc_sh.at[didx_v], add=True)
        return carry

    lax.fori_loop(0, NCHUNK, chunk, 0)
    plsc.subcore_barrier()
    pltpu.sync_copy(acc_sh.at[pl.ds(b, NPS)], out_hbm.at[c, pl.ds(b, NPS)])


_sc_params = pltpu.CompilerParams(use_tc_tiling_on_sc=False,
                                  needs_layout_passes=False)

_sc_scatter = pl.kernel(
    _scat_body,
    out_type=jax.ShapeDtypeStruct((NC, NP, F), jnp.float32),
    mesh=_mesh,
    compiler_params=_sc_params,
    scratch_types=[
        pltpu.VMEM_SHARED((NP, F), jnp.float32),
        pltpu.VMEM_SHARED((NP, F), jnp.float32),
        pltpu.VMEM((CH,), jnp.int32),
        pltpu.VMEM((CH,), jnp.int32),
        pltpu.VMEM((CH, F), jnp.float32),
        pltpu.VMEM((RCH, 128), jnp.float32),
        pltpu.VMEM((CROWS, F), jnp.float32),
        pltpu.SemaphoreType.DMA,
    ],
)


def _deg_body(ei_hbm, zrows_hbm, ones_hbm, out_hbm,
              acc_sh, idx_v, ones_v, pk_c, rows_c, sem):
    c = lax.axis_index("c")
    s = lax.axis_index("s")
    wid = c * NS + s
    b = pl.multiple_of(s * NPS, 8)
    rb = pl.multiple_of(s * RPS, 8)
    pltpu.sync_copy(zrows_hbm, ones_v)
    _zero_acc(b, ones_v, acc_sh)
    pltpu.sync_copy(ones_hbm, ones_v)
    plsc.subcore_barrier()

    def chunk(ci, carry):
        base = wid * EPW + ci * CH
        pltpu.sync_copy(ei_hbm.at[pl.ds(E + base, CH)], idx_v)
        pltpu.sync_copy(ones_v, acc_sh.at[idx_v], add=True)
        return carry

    lax.fori_loop(0, NCHUNK, chunk, 0)
    plsc.subcore_barrier()
    pltpu.sync_copy(acc_sh.at[pl.ds(b, NPS)], out_hbm.at[c, pl.ds(b, NPS)])


_sc_deg = pl.kernel(
    _deg_body,
    out_type=jax.ShapeDtypeStruct((NC, NP, F), jnp.float32),
    mesh=_mesh,
    compiler_params=_sc_params,
    scratch_types=[
        pltpu.VMEM_SHARED((NP, F), jnp.float32),
        pltpu.VMEM((CH,), jnp.int32),
        pltpu.VMEM((CH, F), jnp.float32),
        pltpu.VMEM((RCH, 128), jnp.float32),
        pltpu.VMEM((CROWS, F), jnp.float32),
        pltpu.SemaphoreType.DMA,
    ],
)


def _prep_body(w1_ref, b1_ref, w2_ref, b2_ref, w3_ref, b3_ref,
               wc_ref, bc_ref,
               k1_ref, k2_ref, k3_ref, kc_ref,
               b1p_ref, b2p_ref, b3p_ref, bcp_ref):
    k1_ref[...] = jnp.zeros((PACK * DIN, 128), jnp.float32)
    k2_ref[...] = jnp.zeros((128, 128), jnp.float32)
    k3_ref[...] = jnp.zeros((128, 128), jnp.float32)
    kc_ref[...] = jnp.zeros((128, 128), jnp.float32)
    for m in range(PACK):
        k1_ref[pl.ds(DIN * m, DIN), pl.ds(F * m, F)] = w1_ref[...]
        k2_ref[pl.ds(F * m, F), pl.ds(F * m, F)] = w2_ref[...]
        k3_ref[pl.ds(F * m, F), pl.ds(F * m, 2)] = w3_ref[...]
        kc_ref[pl.ds(F * m, 2), pl.ds(F * m, F)] = wc_ref[...]
        b1p_ref[0, pl.ds(F * m, F)] = b1_ref[...]
        b2p_ref[0, pl.ds(F * m, F)] = b2_ref[...]
        b3p_ref[0, pl.ds(F * m, 2)] = b3_ref[...]
        b3p_ref[0, pl.ds(F * m + 2, 2)] = jnp.zeros((2,), jnp.float32)
        bcp_ref[0, pl.ds(F * m, F)] = bc_ref[...]


def _dense1_body(xp_ref, k1_ref, deg_ref, y1_ref, dinv_ref):
    dinv = lax.rsqrt(deg_ref[0] + deg_ref[1] + 1.0)
    dinv_ref[...] = dinv
    xw = jnp.dot(xp_ref[...], k1_ref[...], preferred_element_type=jnp.float32)
    y1_ref[...] = jnp.concatenate(
        [xw, jnp.zeros((RP - RX, 128), jnp.float32)], axis=0) * dinv


def _dense_mid_body(s_ref, y_ref, dinv_ref, b_ref, k_ref, h_ref, ynext_ref):
    dinv = dinv_ref[...]
    h = jnp.tanh(dinv * (s_ref[0] + s_ref[1] + y_ref[...]) + b_ref[...])
    h_ref[...] = h
    ynext_ref[...] = jnp.dot(h, k_ref[...],
                             preferred_element_type=jnp.float32) * dinv


def _dense_out_body(s_ref, y_ref, dinv_ref, b_ref, kc_ref, bc_ref,
                    h_ref, out_ref):
    dinv = dinv_ref[...]
    h = jnp.tanh(dinv * (s_ref[0] + s_ref[1] + y_ref[...]) + b_ref[...])
    h_ref[...] = h
    out_ref[...] = jnp.dot(h, kc_ref[...],
                           preferred_element_type=jnp.float32) + bc_ref[...]


def _repack_body(o_ref, h1_ref, h2_ref, h3_ref,
                 oo_ref, ho1_ref, ho2_ref, ho3_ref):
    oo_ref[...] = o_ref[...].reshape(256, F)
    ho1_ref[...] = h1_ref[...].reshape(256, F)
    ho2_ref[...] = h2_ref[...].reshape(256, F)
    ho3_ref[...] = h3_ref[...].reshape(256, F)[:, :2]


_p128 = jax.ShapeDtypeStruct((RP, 128), jnp.float32)
_row1 = jax.ShapeDtypeStruct((1, 128), jnp.float32)

_prep = pl.pallas_call(
    _prep_body,
    out_shape=(
        jax.ShapeDtypeStruct((PACK * DIN, 128), jnp.float32),
        jax.ShapeDtypeStruct((128, 128), jnp.float32),
        jax.ShapeDtypeStruct((128, 128), jnp.float32),
        jax.ShapeDtypeStruct((128, 128), jnp.float32),
        _row1, _row1, _row1, _row1,
    ),
)

_dense1 = pl.pallas_call(_dense1_body, out_shape=(_p128, _p128))
_dense_mid = pl.pallas_call(_dense_mid_body, out_shape=(_p128, _p128))
_dense_out = pl.pallas_call(_dense_out_body, out_shape=(_p128, _p128))

_NBLK = 391

_repack = pl.pallas_call(
    _repack_body,
    grid=(_NBLK,),
    in_specs=[pl.BlockSpec((8, 128), lambda i: (i, 0))] * 4,
    out_specs=[
        pl.BlockSpec((256, F), lambda i: (i, 0)),
        pl.BlockSpec((256, F), lambda i: (i, 0)),
        pl.BlockSpec((256, F), lambda i: (i, 0)),
        pl.BlockSpec((256, 2), lambda i: (i, 0)),
    ],
    out_shape=(
        jax.ShapeDtypeStruct((N, F), jnp.float32),
        jax.ShapeDtypeStruct((N, F), jnp.float32),
        jax.ShapeDtypeStruct((N, F), jnp.float32),
        jax.ShapeDtypeStruct((N, 2), jnp.float32),
    ),
)


def kernel(x, edge_index, W1, b1, W2, b2, W3, b3, Wc, bc):
    eif = edge_index.astype(jnp.int32).reshape(2 * E)
    xp = x.reshape(RX, PACK * DIN)
    zrows = jnp.zeros((CH, F), jnp.float32)
    ones = jnp.ones((CH, F), jnp.float32)

    K1, K2, K3, Kc, b1p, b2p, b3p, bcp = _prep(
        W1, b1, W2, b2, W3, b3, Wc, bc)

    deg = _sc_deg(eif, zrows, ones).reshape(NC, RP, 128)
    y1, dinv = _dense1(xp, K1, deg)
    s1 = _sc_scatter(eif, y1, zrows).reshape(NC, RP, 128)
    h1, y2 = _dense_mid(s1, y1, dinv, b1p, K2)
    s2 = _sc_scatter(eif, y2, zrows).reshape(NC, RP, 128)
    h2, y3 = _dense_mid(s2, y2, dinv, b2p, K3)
    s3 = _sc_scatter(eif, y3, zrows).reshape(NC, RP, 128)
    h3, outp = _dense_out(s3, y3, dinv, b3p, Kc, bcp)

    return (outp.reshape(NP, F)[:N],
            h1.reshape(NP, F)[:N],
            h2.reshape(NP, F)[:N],
            h3.reshape(NP, F)[:N, :2])


# speedup vs baseline: 38.0677x; 1.0256x over previous
import functools

import jax
import jax.numpy as jnp
from jax import lax
from jax.experimental import pallas as pl
from jax.experimental.pallas import tpu as pltpu
from jax.experimental.pallas import tpu_sc as plsc

N = 100000
E = 1600000
F = 4
PACK = 32
NP = 102400
RP = NP // PACK
RX = N // PACK
DIN = 34
NC, NS = 2, 16
NW = NC * NS
EPW = E // NW
CH = 2000
NCHUNK = EPW // CH
NPS = NP // NS
RPS = RP // NS
RCH = 8
NCHR = RPS // RCH
CROWS = RCH * PACK

_mesh = plsc.VectorSubcoreMesh(core_axis_name="c", subcore_axis_name="s")


def _iota16():
    i = lax.iota(jnp.int32, 16)
    return i >> 2, i & 3


def _unpack_chunk(pk_c, rows_c):
    r0, c0 = _iota16()
    for r in range(RCH):
        for j in range(8):
            v = pk_c[r, pl.ds(j * 16, 16)]
            plsc.store_scatter(rows_c, [32 * r + 4 * j + r0, c0], v)


def _pack_chunk(rows_c, pk_c):
    r0, c0 = _iota16()
    for r in range(RCH):
        for j in range(8):
            v = plsc.load_gather(rows_c, [32 * r + 4 * j + r0, c0])
            pk_c[r, pl.ds(j * 16, 16)] = v


def _stage_packed_to_rows(src_hbm_rowslice_fn, rows_dst, pk_c, rows_c):
    def stage(q, carry):
        pltpu.sync_copy(src_hbm_rowslice_fn(q), pk_c)
        _unpack_chunk(pk_c, rows_c)
        pltpu.sync_copy(rows_c, rows_dst(q))
        return carry

    lax.fori_loop(0, NCHR, stage, 0)


def _write_rows_to_packed(rows_src, dst_hbm_rowslice_fn, pk_c, rows_c):
    def wb(q, carry):
        pltpu.sync_copy(rows_src(q), rows_c)
        _pack_chunk(rows_c, pk_c)
        pltpu.sync_copy(pk_c, dst_hbm_rowslice_fn(q))
        return carry

    lax.fori_loop(0, NCHR, wb, 0)


def _zero_acc(b, zrows_v, acc_sh):
    for k in range(NPS // CH):
        pltpu.sync_copy(zrows_v, acc_sh.at[pl.ds(b + k * CH, CH)])
    rem = NPS - (NPS // CH) * CH
    if rem:
        pltpu.sync_copy(zrows_v.at[pl.ds(0, rem)],
                        acc_sh.at[pl.ds(b + NPS - rem, rem)])


def _scat_body(ei_hbm, y_hbm, zrows_hbm, out_hbm,
               y_sh, acc_sh, sidx_v, didx_v, rows_v, pk_c, rows_c, sem):
    c = lax.axis_index("c")
    s = lax.axis_index("s")
    wid = c * NS + s
    b = pl.multiple_of(s * NPS, 8)
    rb = pl.multiple_of(s * RPS, 8)
    pltpu.sync_copy(zrows_hbm, rows_v)
    _zero_acc(b, rows_v, acc_sh)
    _stage_packed_to_rows(
        lambda q: y_hbm.at[pl.ds(rb + q * RCH, RCH)],
        lambda q: y_sh.at[pl.ds(b + q * CROWS, CROWS)],
        pk_c, rows_c)
    plsc.subcore_barrier()

    def chunk(ci, carry):
        base = wid * EPW + ci * CH
        pltpu.sync_copy(ei_hbm.at[pl.ds(base, CH)], sidx_v)
        pltpu.sync_copy(ei_hbm.at[pl.ds(E + base, CH)], didx_v)
        pltpu.async_copy(y_sh.at[sidx_v], rows_v, sem).wait()
        pltpu.sync_copy(rows_v, acc_sh.at[didx_v], add=True)
        return carry

    lax.fori_loop(0, NCHUNK, chunk, 0)
    plsc.subcore_barrier()
    pltpu.sync_copy(acc_sh.at[pl.ds(b, NPS)], out_hbm.at[c, pl.ds(b, NPS)])


_sc_params = pltpu.CompilerParams(use_tc_tiling_on_sc=False,
                                  needs_layout_passes=False)

_sc_scatter = pl.kernel(
    _scat_body,
    out_type=jax.ShapeDtypeStruct((NC, NP, F), jnp.float32),
    mesh=_mesh,
    compiler_params=_sc_params,
    scratch_types=[
        pltpu.VMEM_SHARED((NP, F), jnp.float32),
        pltpu.VMEM_SHARED((NP, F), jnp.float32),
        pltpu.VMEM((CH,), jnp.int32),
        pltpu.VMEM((CH,), jnp.int32),
        pltpu.VMEM((CH, F), jnp.float32),
        pltpu.VMEM((RCH, 128), jnp.float32),
        pltpu.VMEM((CROWS, F), jnp.float32),
        pltpu.SemaphoreType.DMA,
    ],
)


def _deg_body(ei_hbm, zrows_hbm, ones_hbm, out_hbm,
              acc_sh, idx_v, ones_v, pk_c, rows_c, sem):
    c = lax.axis_index("c")
    s = lax.axis_index("s")
    wid = c * NS + s
    b = pl.multiple_of(s * NPS, 8)
    rb = pl.multiple_of(s * RPS, 8)
    pltpu.sync_copy(zrows_hbm, ones_v)
    _zero_acc(b, ones_v, acc_sh)
    pltpu.sync_copy(ones_hbm, ones_v)
    plsc.subcore_barrier()

    def chunk(ci, carry):
        base = wid * EPW + ci * CH
        pltpu.sync_copy(ei_hbm.at[pl.ds(E + base, CH)], idx_v)
        pltpu.sync_copy(ones_v, acc_sh.at[idx_v], add=True)
        return carry

    lax.fori_loop(0, NCHUNK, chunk, 0)
    plsc.subcore_barrier()
    pltpu.sync_copy(acc_sh.at[pl.ds(b, NPS)], out_hbm.at[c, pl.ds(b, NPS)])


_sc_deg = pl.kernel(
    _deg_body,
    out_type=jax.ShapeDtypeStruct((NC, NP, F), jnp.float32),
    mesh=_mesh,
    compiler_params=_sc_params,
    scratch_types=[
        pltpu.VMEM_SHARED((NP, F), jnp.float32),
        pltpu.VMEM((CH,), jnp.int32),
        pltpu.VMEM((CH, F), jnp.float32),
        pltpu.VMEM((RCH, 128), jnp.float32),
        pltpu.VMEM((CROWS, F), jnp.float32),
        pltpu.SemaphoreType.DMA,
    ],
)


def _prep_body(w1_ref, b1_ref, w2_ref, b2_ref, w3_ref, b3_ref,
               wc_ref, bc_ref,
               k1_ref, k2_ref, k3_ref, kc_ref,
               b1p_ref, b2p_ref, b3p_ref, bcp_ref):
    k1_ref[...] = jnp.zeros((PACK * DIN, 128), jnp.float32)
    k2_ref[...] = jnp.zeros((128, 128), jnp.float32)
    k3_ref[...] = jnp.zeros((128, 128), jnp.float32)
    kc_ref[...] = jnp.zeros((128, 128), jnp.float32)
    for m in range(PACK):
        k1_ref[pl.ds(DIN * m, DIN), pl.ds(F * m, F)] = w1_ref[...]
        k2_ref[pl.ds(F * m, F), pl.ds(F * m, F)] = w2_ref[...]
        k3_ref[pl.ds(F * m, F), pl.ds(F * m, 2)] = w3_ref[...]
        kc_ref[pl.ds(F * m, 2), pl.ds(F * m, F)] = wc_ref[...]
        b1p_ref[0, pl.ds(F * m, F)] = b1_ref[...]
        b2p_ref[0, pl.ds(F * m, F)] = b2_ref[...]
        b3p_ref[0, pl.ds(F * m, 2)] = b3_ref[...]
        b3p_ref[0, pl.ds(F * m + 2, 2)] = jnp.zeros((2,), jnp.float32)
        bcp_ref[0, pl.ds(F * m, F)] = bc_ref[...]


def _dense1_body(xp_ref, k1_ref, deg_ref, y1_ref, dinv_ref):
    dinv = lax.rsqrt(deg_ref[0] + deg_ref[1] + 1.0)
    dinv_ref[...] = dinv
    xw = jnp.dot(xp_ref[...], k1_ref[...], preferred_element_type=jnp.float32)
    y1_ref[...] = jnp.concatenate(
        [xw, jnp.zeros((RP - RX, 128), jnp.float32)], axis=0) * dinv


def _dense_mid_body(s_ref, y_ref, dinv_ref, b_ref, k_ref, h_ref, ynext_ref):
    dinv = dinv_ref[...]
    h = jnp.tanh(dinv * (s_ref[0] + s_ref[1] + y_ref[...]) + b_ref[...])
    h_ref[...] = h[:RX]
    ynext_ref[...] = jnp.dot(h, k_ref[...],
                             preferred_element_type=jnp.float32) * dinv


def _dense_out_body(s_ref, y_ref, dinv_ref, b_ref, kc_ref, bc_ref,
                    h_ref, out_ref):
    dinv = dinv_ref[...]
    h = jnp.tanh(dinv * (s_ref[0] + s_ref[1] + y_ref[...]) + b_ref[...])
    h_ref[...] = h[:RX]
    out_ref[...] = jnp.dot(h[:RX], kc_ref[...],
                           preferred_element_type=jnp.float32) + bc_ref[...]


def _repack_body(o_ref, h1_ref, h2_ref, h3_ref,
                 oo_ref, ho1_ref, ho2_ref, ho3_ref):
    oo_ref[...] = o_ref[...].reshape(256, F)
    ho1_ref[...] = h1_ref[...].reshape(256, F)
    ho2_ref[...] = h2_ref[...].reshape(256, F)
    ho3_ref[...] = h3_ref[...].reshape(256, F)[:, :2]


_p128 = jax.ShapeDtypeStruct((RP, 128), jnp.float32)
_row1 = jax.ShapeDtypeStruct((1, 128), jnp.float32)

_prep = pl.pallas_call(
    _prep_body,
    out_shape=(
        jax.ShapeDtypeStruct((PACK * DIN, 128), jnp.float32),
        jax.ShapeDtypeStruct((128, 128), jnp.float32),
        jax.ShapeDtypeStruct((128, 128), jnp.float32),
        jax.ShapeDtypeStruct((128, 128), jnp.float32),
        _row1, _row1, _row1, _row1,
    ),
)

_x128 = jax.ShapeDtypeStruct((RX, 128), jnp.float32)

_dense1 = pl.pallas_call(_dense1_body, out_shape=(_p128, _p128))
_dense_mid = pl.pallas_call(_dense_mid_body, out_shape=(_x128, _p128))
_dense_out = pl.pallas_call(_dense_out_body, out_shape=(_x128, _x128))

_NBLK = 391

_repack = pl.pallas_call(
    _repack_body,
    grid=(_NBLK,),
    in_specs=[pl.BlockSpec((8, 128), lambda i: (i, 0))] * 4,
    out_specs=[
        pl.BlockSpec((256, F), lambda i: (i, 0)),
        pl.BlockSpec((256, F), lambda i: (i, 0)),
        pl.BlockSpec((256, F), lambda i: (i, 0)),
        pl.BlockSpec((256, 2), lambda i: (i, 0)),
    ],
    out_shape=(
        jax.ShapeDtypeStruct((N, F), jnp.float32),
        jax.ShapeDtypeStruct((N, F), jnp.float32),
        jax.ShapeDtypeStruct((N, F), jnp.float32),
        jax.ShapeDtypeStruct((N, 2), jnp.float32),
    ),
)


def kernel(x, edge_index, W1, b1, W2, b2, W3, b3, Wc, bc):
    eif = edge_index.astype(jnp.int32).reshape(2 * E)
    xp = x.reshape(RX, PACK * DIN)
    zrows = jnp.zeros((CH, F), jnp.float32)
    ones = jnp.ones((CH, F), jnp.float32)

    K1, K2, K3, Kc, b1p, b2p, b3p, bcp = _prep(
        W1, b1, W2, b2, W3, b3, Wc, bc)

    deg = _sc_deg(eif, zrows, ones).reshape(NC, RP, 128)
    y1, dinv = _dense1(xp, K1, deg)
    s1 = _sc_scatter(eif, y1, zrows).reshape(NC, RP, 128)
    h1, y2 = _dense_mid(s1, y1, dinv, b1p, K2)
    s2 = _sc_scatter(eif, y2, zrows).reshape(NC, RP, 128)
    h2, y3 = _dense_mid(s2, y2, dinv, b2p, K3)
    s3 = _sc_scatter(eif, y3, zrows).reshape(NC, RP, 128)
    h3, outp = _dense_out(s3, y3, dinv, b3p, Kc, bcp)

    return (outp.reshape(N, F),
            h1.reshape(N, F),
            h2.reshape(N, F),
            h3.reshape(N, F)[:, :2])


# speedup vs baseline: 66.9432x; 1.7585x over previous
import functools

import jax
import jax.numpy as jnp
from jax import lax
from jax.experimental import pallas as pl
from jax.experimental.pallas import tpu as pltpu
from jax.experimental.pallas import tpu_sc as plsc

N = 100000
E = 1600000
F = 4
PACK = 32
NP = 102400
RP = NP // PACK
RX = N // PACK
DIN = 34
NC, NS = 2, 16
NW = NC * NS
EPW = E // NW
CH = 2000
NCHUNK = EPW // CH
NPS = NP // NS
RPS = RP // NS
RCH = 8
NCHR = RPS // RCH
CROWS = RCH * PACK

_mesh = plsc.VectorSubcoreMesh(core_axis_name="c", subcore_axis_name="s")


def _iota16():
    i = lax.iota(jnp.int32, 16)
    return i >> 2, i & 3


def _unpack_chunk(pk_c, rows_c):
    r0, c0 = _iota16()
    for r in range(RCH):
        for j in range(8):
            v = pk_c[r, pl.ds(j * 16, 16)]
            plsc.store_scatter(rows_c, [32 * r + 4 * j + r0, c0], v)


def _pack_chunk(rows_c, pk_c):
    i = lax.iota(jnp.int32, 16)
    r0, c0 = i >> 2, i & 3
    for r in range(RCH):
        rr = jnp.full((16,), r, jnp.int32)
        for j in range(8):
            v = plsc.load_gather(rows_c, [32 * r + 4 * j + r0, c0])
            plsc.store_scatter(pk_c, [rr, j * 16 + i], v)


def _stage_packed_to_rows(src_hbm_rowslice_fn, rows_dst, pk_c, rows_c):
    def stage(q, carry):
        pltpu.sync_copy(src_hbm_rowslice_fn(q), pk_c)
        _unpack_chunk(pk_c, rows_c)
        pltpu.sync_copy(rows_c, rows_dst(q))
        return carry

    lax.fori_loop(0, NCHR, stage, 0)


def _write_rows_to_packed(rows_src, dst_hbm_rowslice_fn, pk_c, rows_c):
    def wb(q, carry):
        pltpu.sync_copy(rows_src(q), rows_c)
        _pack_chunk(rows_c, pk_c)
        pltpu.sync_copy(pk_c, dst_hbm_rowslice_fn(q))
        return carry

    lax.fori_loop(0, NCHR, wb, 0)


def _zero_acc(b, zrows_v, acc_sh):
    for k in range(NPS // CH):
        pltpu.sync_copy(zrows_v, acc_sh.at[pl.ds(b + k * CH, CH)])
    rem = NPS - (NPS // CH) * CH
    if rem:
        pltpu.sync_copy(zrows_v.at[pl.ds(0, rem)],
                        acc_sh.at[pl.ds(b + NPS - rem, rem)])


def _scat_body(ei_hbm, y_hbm, zrows_hbm, out_hbm,
               y_sh, acc_sh, sidx_v, didx_v, rows_v, pk_c, rows_c, sem):
    c = lax.axis_index("c")
    s = lax.axis_index("s")
    wid = c * NS + s
    b = pl.multiple_of(s * NPS, 8)
    rb = pl.multiple_of(s * RPS, 8)
    pltpu.sync_copy(zrows_hbm, rows_v)
    _zero_acc(b, rows_v, acc_sh)
    _stage_packed_to_rows(
        lambda q: y_hbm.at[pl.ds(rb + q * RCH, RCH)],
        lambda q: y_sh.at[pl.ds(b + q * CROWS, CROWS)],
        pk_c, rows_c)
    plsc.subcore_barrier()

    def chunk(ci, carry):
        base = wid * EPW + ci * CH
        pltpu.sync_copy(ei_hbm.at[pl.ds(base, CH)], sidx_v)
        pltpu.sync_copy(ei_hbm.at[pl.ds(E + base, CH)], didx_v)
        pltpu.async_copy(y_sh.at[sidx_v], rows_v, sem).wait()
        pltpu.sync_copy(rows_v, acc_sh.at[didx_v], add=True)
        return carry

    lax.fori_loop(0, NCHUNK, chunk, 0)
    plsc.subcore_barrier()
    _write_rows_to_packed(
        lambda q: acc_sh.at[pl.ds(b + q * CROWS, CROWS)],
        lambda q: out_hbm.at[c, pl.ds(rb + q * RCH, RCH)],
        pk_c, rows_c)


_sc_params = pltpu.CompilerParams(use_tc_tiling_on_sc=False,
                                  needs_layout_passes=False)

_sc_scatter = pl.kernel(
    _scat_body,
    out_type=jax.ShapeDtypeStruct((NC, RP, 128), jnp.float32),
    mesh=_mesh,
    compiler_params=_sc_params,
    scratch_types=[
        pltpu.VMEM_SHARED((NP, F), jnp.float32),
        pltpu.VMEM_SHARED((NP, F), jnp.float32),
        pltpu.VMEM((CH,), jnp.int32),
        pltpu.VMEM((CH,), jnp.int32),
        pltpu.VMEM((CH, F), jnp.float32),
        pltpu.VMEM((RCH, 128), jnp.float32),
        pltpu.VMEM((CROWS, F), jnp.float32),
        pltpu.SemaphoreType.DMA,
    ],
)


def _deg_body(ei_hbm, zrows_hbm, ones_hbm, out_hbm,
              acc_sh, idx_v, ones_v, pk_c, rows_c, sem):
    c = lax.axis_index("c")
    s = lax.axis_index("s")
    wid = c * NS + s
    b = pl.multiple_of(s * NPS, 8)
    rb = pl.multiple_of(s * RPS, 8)
    pltpu.sync_copy(zrows_hbm, ones_v)
    _zero_acc(b, ones_v, acc_sh)
    pltpu.sync_copy(ones_hbm, ones_v)
    plsc.subcore_barrier()

    def chunk(ci, carry):
        base = wid * EPW + ci * CH
        pltpu.sync_copy(ei_hbm.at[pl.ds(E + base, CH)], idx_v)
        pltpu.sync_copy(ones_v, acc_sh.at[idx_v], add=True)
        return carry

    lax.fori_loop(0, NCHUNK, chunk, 0)
    plsc.subcore_barrier()
    _write_rows_to_packed(
        lambda q: acc_sh.at[pl.ds(b + q * CROWS, CROWS)],
        lambda q: out_hbm.at[c, pl.ds(rb + q * RCH, RCH)],
        pk_c, rows_c)


_sc_deg = pl.kernel(
    _deg_body,
    out_type=jax.ShapeDtypeStruct((NC, RP, 128), jnp.float32),
    mesh=_mesh,
    compiler_params=_sc_params,
    scratch_types=[
        pltpu.VMEM_SHARED((NP, F), jnp.float32),
        pltpu.VMEM((CH,), jnp.int32),
        pltpu.VMEM((CH, F), jnp.float32),
        pltpu.VMEM((RCH, 128), jnp.float32),
        pltpu.VMEM((CROWS, F), jnp.float32),
        pltpu.SemaphoreType.DMA,
    ],
)


def _prep_body(w1_ref, b1_ref, w2_ref, b2_ref, w3_ref, b3_ref,
               wc_ref, bc_ref,
               k1_ref, k2_ref, k3_ref, kc_ref,
               b1p_ref, b2p_ref, b3p_ref, bcp_ref):
    k1_ref[...] = jnp.zeros((PACK * DIN, 128), jnp.float32)
    k2_ref[...] = jnp.zeros((128, 128), jnp.float32)
    k3_ref[...] = jnp.zeros((128, 128), jnp.float32)
    kc_ref[...] = jnp.zeros((128, 128), jnp.float32)
    for m in range(PACK):
        k1_ref[pl.ds(DIN * m, DIN), pl.ds(F * m, F)] = w1_ref[...]
        k2_ref[pl.ds(F * m, F), pl.ds(F * m, F)] = w2_ref[...]
        k3_ref[pl.ds(F * m, F), pl.ds(F * m, 2)] = w3_ref[...]
        kc_ref[pl.ds(F * m, 2), pl.ds(F * m, F)] = wc_ref[...]
        b1p_ref[0, pl.ds(F * m, F)] = b1_ref[...]
        b2p_ref[0, pl.ds(F * m, F)] = b2_ref[...]
        b3p_ref[0, pl.ds(F * m, 2)] = b3_ref[...]
        b3p_ref[0, pl.ds(F * m + 2, 2)] = jnp.zeros((2,), jnp.float32)
        bcp_ref[0, pl.ds(F * m, F)] = bc_ref[...]


def _dense1_body(xp_ref, k1_ref, deg_ref, y1_ref, dinv_ref):
    dinv = lax.rsqrt(deg_ref[0] + deg_ref[1] + 1.0)
    dinv_ref[...] = dinv
    xw = jnp.dot(xp_ref[...], k1_ref[...], preferred_element_type=jnp.float32)
    y1_ref[...] = jnp.concatenate(
        [xw, jnp.zeros((RP - RX, 128), jnp.float32)], axis=0) * dinv


def _dense_mid_body(s_ref, y_ref, dinv_ref, b_ref, k_ref, h_ref, ynext_ref):
    dinv = dinv_ref[...]
    h = jnp.tanh(dinv * (s_ref[0] + s_ref[1] + y_ref[...]) + b_ref[...])
    h_ref[...] = h[:RX]
    ynext_ref[...] = jnp.dot(h, k_ref[...],
                             preferred_element_type=jnp.float32) * dinv


def _dense_out_body(s_ref, y_ref, dinv_ref, b_ref, kc_ref, bc_ref,
                    h_ref, out_ref):
    dinv = dinv_ref[...]
    h = jnp.tanh(dinv * (s_ref[0] + s_ref[1] + y_ref[...]) + b_ref[...])
    h_ref[...] = h[:RX]
    out_ref[...] = jnp.dot(h[:RX], kc_ref[...],
                           preferred_element_type=jnp.float32) + bc_ref[...]


def _repack_body(o_ref, h1_ref, h2_ref, h3_ref,
                 oo_ref, ho1_ref, ho2_ref, ho3_ref):
    oo_ref[...] = o_ref[...].reshape(256, F)
    ho1_ref[...] = h1_ref[...].reshape(256, F)
    ho2_ref[...] = h2_ref[...].reshape(256, F)
    ho3_ref[...] = h3_ref[...].reshape(256, F)[:, :2]


_p128 = jax.ShapeDtypeStruct((RP, 128), jnp.float32)
_row1 = jax.ShapeDtypeStruct((1, 128), jnp.float32)

_prep = pl.pallas_call(
    _prep_body,
    out_shape=(
        jax.ShapeDtypeStruct((PACK * DIN, 128), jnp.float32),
        jax.ShapeDtypeStruct((128, 128), jnp.float32),
        jax.ShapeDtypeStruct((128, 128), jnp.float32),
        jax.ShapeDtypeStruct((128, 128), jnp.float32),
        _row1, _row1, _row1, _row1,
    ),
)

_x128 = jax.ShapeDtypeStruct((RX, 128), jnp.float32)

_dense1 = pl.pallas_call(_dense1_body, out_shape=(_p128, _p128))
_dense_mid = pl.pallas_call(_dense_mid_body, out_shape=(_x128, _p128))
_dense_out = pl.pallas_call(_dense_out_body, out_shape=(_x128, _x128))

_NBLK = 391

_repack = pl.pallas_call(
    _repack_body,
    grid=(_NBLK,),
    in_specs=[pl.BlockSpec((8, 128), lambda i: (i, 0))] * 4,
    out_specs=[
        pl.BlockSpec((256, F), lambda i: (i, 0)),
        pl.BlockSpec((256, F), lambda i: (i, 0)),
        pl.BlockSpec((256, F), lambda i: (i, 0)),
        pl.BlockSpec((256, 2), lambda i: (i, 0)),
    ],
    out_shape=(
        jax.ShapeDtypeStruct((N, F), jnp.float32),
        jax.ShapeDtypeStruct((N, F), jnp.float32),
        jax.ShapeDtypeStruct((N, F), jnp.float32),
        jax.ShapeDtypeStruct((N, 2), jnp.float32),
    ),
)


def kernel(x, edge_index, W1, b1, W2, b2, W3, b3, Wc, bc):
    eif = edge_index.astype(jnp.int32).reshape(2 * E)
    xp = x.reshape(RX, PACK * DIN)
    zrows = jnp.zeros((CH, F), jnp.float32)
    ones = jnp.ones((CH, F), jnp.float32)

    K1, K2, K3, Kc, b1p, b2p, b3p, bcp = _prep(
        W1, b1, W2, b2, W3, b3, Wc, bc)

    deg = _sc_deg(eif, zrows, ones)
    y1, dinv = _dense1(xp, K1, deg)
    s1 = _sc_scatter(eif, y1, zrows)
    h1, y2 = _dense_mid(s1, y1, dinv, b1p, K2)
    s2 = _sc_scatter(eif, y2, zrows)
    h2, y3 = _dense_mid(s2, y2, dinv, b2p, K3)
    s3 = _sc_scatter(eif, y3, zrows)
    h3, outp = _dense_out(s3, y3, dinv, b3p, Kc, bcp)

    return (outp.reshape(N, F),
            h1.reshape(N, F),
            h2.reshape(N, F),
            h3.reshape(N, F)[:, :2])
